# Initial kernel scaffold; baseline (speedup 1.0000x reference)
import jax
import jax.numpy as jnp
from jax.experimental import pallas as pl


def kernel(edge_weights, features, W1, b1, gamma1, beta1, W2, b2, gamma2, beta2, Wih, Whh, bih, bhh, lin1_W, lin1_b, lin2_W, lin2_b):
    raise NotImplementedError("write your pallas kernel here")


# speedup vs baseline: 1189.1382x; 1189.1382x over previous
import jax
import jax.numpy as jnp
from jax import lax
from jax.experimental import pallas as pl

_N = 1024
_DH = 64
_STEPS = 10

_T_DIMS = (((0,), (0,)), ((), ()))
_C_DIMS = (((1,), (1,)), ((), ()))


def _leaky(x):
    return jnp.where(x > 0, x, 0.01 * x)


def _body(E_ref, x_ref, W1_ref, b1_ref, ga1_ref, be1_ref,
          W2_ref, b2_ref, ga2_ref, be2_ref,
          Wii_ref, Wif_ref, Wig_ref, Wio_ref,
          Whi_ref, Whf_ref, Whg_ref, Who_ref,
          bi_ref, bf_ref, bg_ref, bo_ref,
          L1_ref, l1b_ref, L2_ref, l2b_ref, out_ref):
    f32 = jnp.float32
    E = E_ref[...]

    ones = jnp.ones((_N, 1), f32)
    deg = lax.dot_general(E, ones, _T_DIMS, preferred_element_type=f32) + 1.0
    dinv = jnp.where(deg > 0, lax.rsqrt(deg), 0.0)

    def gcn(xin, W_ref, b_ref):
        y = dinv * jnp.dot(xin, W_ref[...], preferred_element_type=f32)
        z = lax.dot_general(E, y, _T_DIMS, preferred_element_type=f32) + y
        return dinv * z + b_ref[...]

    def bn(v, g_ref, b_ref):
        m = jnp.mean(v, axis=0, keepdims=True)
        var = jnp.mean((v - m) ** 2, axis=0, keepdims=True)
        return g_ref[...] * (v - m) * lax.rsqrt(var + 1e-5) + b_ref[...]

    h1 = _leaky(bn(gcn(x_ref[...], W1_ref, b1_ref), ga1_ref, be1_ref))
    x2 = _leaky(bn(gcn(h1, W2_ref, b2_ref), ga2_ref, be2_ref) + h1)

    h = jnp.zeros((1, _DH), f32)
    c = jnp.zeros((1, _DH), f32)
    q_star = jnp.zeros((1, 2 * _DH), f32)
    for _ in range(_STEPS):
        def gate(Wi_ref, Wh_ref, b_ref):
            return (lax.dot_general(q_star, Wi_ref[...], _C_DIMS,
                                    preferred_element_type=f32)
                    + lax.dot_general(h, Wh_ref[...], _C_DIMS,
                                      preferred_element_type=f32)
                    + b_ref[...])
        i = jax.nn.sigmoid(gate(Wii_ref, Whi_ref, bi_ref))
        f = jax.nn.sigmoid(gate(Wif_ref, Whf_ref, bf_ref))
        g = jnp.tanh(gate(Wig_ref, Whg_ref, bg_ref))
        o = jax.nn.sigmoid(gate(Wio_ref, Who_ref, bo_ref))
        c = f * c + i * g
        h = o * jnp.tanh(c)
        e = lax.dot_general(x2, h, _C_DIMS, preferred_element_type=f32)
        a = jnp.exp(e - jnp.max(e, axis=0, keepdims=True))
        a = a / jnp.sum(a, axis=0, keepdims=True)
        r = lax.dot_general(a, x2, _T_DIMS, preferred_element_type=f32)
        q_star = jnp.concatenate([h, r], axis=1)

    o1 = _leaky(jnp.dot(q_star, L1_ref[...], preferred_element_type=f32)
                + l1b_ref[...])
    out_ref[...] = (jnp.dot(o1, L2_ref[...], preferred_element_type=f32)
                    + l2b_ref[...])


def _call(*args):
    return pl.pallas_call(
        _body,
        out_shape=jax.ShapeDtypeStruct((1, 16), jnp.float32),
    )(*args)


def _prep(edge_weights, features, W1, b1, gamma1, beta1, W2, b2, gamma2,
          beta2, Wih, Whh, bih, bhh, lin1_W, lin1_b, lin2_W, lin2_b):
    Wii, Wif, Wig, Wio = jnp.split(Wih, 4, axis=0)
    Whi, Whf, Whg, Who = jnp.split(Whh, 4, axis=0)
    bsum = (bih + bhh).reshape(1, -1)
    bi, bf, bg, bo = jnp.split(bsum, 4, axis=1)
    row = lambda v: v.reshape(1, -1)
    return (edge_weights, features, W1, row(b1), row(gamma1), row(beta1),
            W2, row(b2), row(gamma2), row(beta2),
            Wii, Wif, Wig, Wio, Whi, Whf, Whg, Who,
            bi, bf, bg, bo, lin1_W, row(lin1_b), lin2_W, row(lin2_b))


def kernel(edge_weights, features, W1, b1, gamma1, beta1, W2, b2, gamma2,
           beta2, Wih, Whh, bih, bhh, lin1_W, lin1_b, lin2_W, lin2_b):
    args = _prep(edge_weights, features, W1, b1, gamma1, beta1, W2, b2,
                 gamma2, beta2, Wih, Whh, bih, bhh, lin1_W, lin1_b,
                 lin2_W, lin2_b)
    return _call(*args)


# speedup vs baseline: 1394.9602x; 1.1731x over previous
import jax
import jax.numpy as jnp
from jax import lax
from jax.experimental import pallas as pl

_N = 1024
_DH = 64
_STEPS = 10

_T_DIMS = (((0,), (0,)), ((), ()))
_C_DIMS = (((1,), (1,)), ((), ()))


def _leaky(x):
    return jnp.maximum(x, 0.01 * x)


def _body(E_ref, x_ref, W1_ref, b1_ref, ga1_ref, be1_ref,
          W2_ref, b2_ref, ga2_ref, be2_ref,
          Wih_ref, Whh_ref, bg_ref,
          L1_ref, l1b_ref, L2_ref, l2b_ref, out_ref):
    f32 = jnp.float32
    E = E_ref[...]

    ones = jnp.ones((_N, 1), f32)
    deg = lax.dot_general(E, ones, _T_DIMS, preferred_element_type=f32) + 1.0
    dinv = jnp.where(deg > 0, lax.rsqrt(deg), 0.0)

    def gcn(xin, W_ref, b_ref):
        y = dinv * jnp.dot(xin, W_ref[...], preferred_element_type=f32)
        z = lax.dot_general(E, y, _T_DIMS, preferred_element_type=f32) + y
        return dinv * z + b_ref[...]

    def bn(v, g_ref, b_ref):
        m = jnp.mean(v, axis=0, keepdims=True)
        var = jnp.mean((v - m) ** 2, axis=0, keepdims=True)
        return g_ref[...] * (v - m) * lax.rsqrt(var + 1e-5) + b_ref[...]

    h1 = _leaky(bn(gcn(x_ref[...], W1_ref, b1_ref), ga1_ref, be1_ref))
    x2 = _leaky(bn(gcn(h1, W2_ref, b2_ref), ga2_ref, be2_ref) + h1)

    Wih = Wih_ref[...]
    Whh = Whh_ref[...]
    bgates = bg_ref[...]
    h = jnp.zeros((1, _DH), f32)
    c = jnp.zeros((1, _DH), f32)
    q_star = jnp.zeros((1, 2 * _DH), f32)
    for _ in range(_STEPS):
        gates = (lax.dot_general(q_star, Wih, _C_DIMS,
                                 preferred_element_type=f32)
                 + lax.dot_general(h, Whh, _C_DIMS,
                                   preferred_element_type=f32)
                 + bgates)
        i = jax.nn.sigmoid(gates[:, 0:_DH])
        f = jax.nn.sigmoid(gates[:, _DH:2 * _DH])
        g = jnp.tanh(gates[:, 2 * _DH:3 * _DH])
        o = jax.nn.sigmoid(gates[:, 3 * _DH:4 * _DH])
        c = f * c + i * g
        h = o * jnp.tanh(c)
        e = lax.dot_general(h, x2, _C_DIMS,
                            preferred_element_type=f32)
        a = jnp.exp(e - jnp.max(e, axis=1, keepdims=True))
        a = a / jnp.sum(a, axis=1, keepdims=True)
        r = jnp.dot(a, x2, preferred_element_type=f32)
        q_star = jnp.concatenate([h, r], axis=1)

    o1 = _leaky(jnp.dot(q_star, L1_ref[...], preferred_element_type=f32)
                + l1b_ref[...])
    out_ref[...] = (jnp.dot(o1, L2_ref[...], preferred_element_type=f32)
                    + l2b_ref[...])


def _call(*args):
    return pl.pallas_call(
        _body,
        out_shape=jax.ShapeDtypeStruct((1, 16), jnp.float32),
    )(*args)


def kernel(edge_weights, features, W1, b1, gamma1, beta1, W2, b2, gamma2,
           beta2, Wih, Whh, bih, bhh, lin1_W, lin1_b, lin2_W, lin2_b):
    row = lambda v: v.reshape(1, -1)
    args = (edge_weights, features, W1, row(b1), row(gamma1), row(beta1),
            W2, row(b2), row(gamma2), row(beta2),
            Wih, Whh, row(bih + bhh),
            lin1_W, row(lin1_b), lin2_W, row(lin2_b))
    return _call(*args)
